# Initial kernel scaffold; baseline (speedup 1.0000x reference)
import jax
import jax.numpy as jnp
from jax.experimental import pallas as pl


def kernel(src_id, tgt_id, emb_src, emb_tgt, W_enc, b_enc, W_dec):
    raise NotImplementedError("write your pallas kernel here")


# speedup vs baseline: 2.0532x; 2.0532x over previous
import functools

import jax
import jax.numpy as jnp
from jax import lax
from jax.experimental import pallas as pl
from jax.experimental.pallas import tpu as pltpu
from jax.experimental.pallas import tpu_sc as plsc

VOCAB = 100000
DIM = 300
BATCH = 16384

_NC = 2
_NS = 16
_NW = _NC * _NS
_BPW = BATCH // _NW
_CH = 128
_NCH = _BPW // _CH
_KG = 16
_NG = _CH // _KG
_MAIN = 256
_TAIL = DIM - _MAIN
_SW = 384

_mesh = plsc.VectorSubcoreMesh(core_axis_name="c", subcore_axis_name="s")


@functools.partial(
    pl.kernel,
    mesh=_mesh,
    out_type=jax.ShapeDtypeStruct((2 * BATCH, _SW), jnp.float32),
    scratch_types=[
        pltpu.VMEM((_CH,), jnp.int32),
        pltpu.VMEM((_CH, _MAIN), jnp.float32),
        pltpu.VMEM((_KG, 8, _TAIL), jnp.float32),
        pltpu.VMEM((_CH, 128), jnp.float32),
        pltpu.SemaphoreType.DMA,
        pltpu.SemaphoreType.DMA,
    ],
)
def _sc_gather(src_id, tgt_id, src_tab, tgt_tab, out_hbm,
               idx_v, main_v, blk_v, tail_v, sem_m, sem_t):
    wid = lax.axis_index("s") * _NC + lax.axis_index("c")
    base = wid * _BPW
    for tsel in range(2):
        id_hbm = src_id if tsel == 0 else tgt_id
        tab = src_tab if tsel == 0 else tgt_tab
        obase = tsel * BATCH + base
        for j in range(_NCH):
            pltpu.sync_copy(id_hbm.at[pl.ds(base + j * _CH, _CH)], idx_v)
            main_cp = pltpu.async_copy(
                tab.at[idx_v, pl.ds(0, _MAIN)], main_v, sem_m)

            def tail_group(g, _):
                gidx = idx_v[pl.ds(g * _KG, _KG)]
                for k in range(_KG):
                    idx = gidx[k]
                    i8 = pl.multiple_of((idx // 8) * 8, 8)
                    pltpu.async_copy(
                        tab.at[pl.ds(i8, 8), pl.ds(_MAIN, _TAIL)],
                        blk_v.at[k], sem_t)
                for k in range(_KG):
                    pltpu.make_async_copy(
                        tab.at[pl.ds(0, 8), pl.ds(_MAIN, _TAIL)],
                        blk_v.at[k], sem_t).wait()
                for k in range(_KG):
                    idx = gidx[k]
                    r = lax.rem(idx, 8)
                    row = g * _KG + k
                    tail_v[row, pl.ds(0, 16)] = blk_v[k, r, pl.ds(0, 16)]
                    tail_v[row, pl.ds(16, 16)] = blk_v[k, r, pl.ds(16, 16)]
                    tail_v[row, pl.ds(_TAIL - 16, 16)] = (
                        blk_v[k, r, pl.ds(_TAIL - 16, 16)])
                return 0

            lax.fori_loop(0, _NG, tail_group, 0)
            main_cp.wait()
            ob = obase + j * _CH
            pltpu.sync_copy(main_v, out_hbm.at[pl.ds(ob, _CH), pl.ds(0, _MAIN)])
            pltpu.sync_copy(tail_v,
                            out_hbm.at[pl.ds(ob, _CH), pl.ds(_MAIN, 128)])


_BLK = 2048
_NBLK = (2 * BATCH) // _BLK
_NTOP = BATCH // _BLK


def _tc_body(x_ref, we_ref, b_ref, wd_ref, o_ref, wc_ref, bc_ref):
    i = pl.program_id(0)

    @pl.when(i == 0)
    def _():
        wc_ref[...] = lax.dot_general(
            we_ref[...], wd_ref[...], (((0,), (0,)), ((), ())),
            preferred_element_type=jnp.float32)
        bc_ref[...] = lax.dot_general(
            b_ref[...], wd_ref[...], (((1,), (0,)), ((), ())),
            preferred_element_type=jnp.float32)

    @pl.when(i < _NTOP)
    def _():
        o_ref[...] = lax.dot_general(
            x_ref[:, :DIM], wc_ref[...], (((1,), (0,)), ((), ())),
            preferred_element_type=jnp.float32) + bc_ref[...]

    @pl.when(i >= _NTOP)
    def _():
        o_ref[...] = x_ref[:, :DIM]


def _tc_transform(staging, W_enc, b2, W_dec):
    return pl.pallas_call(
        _tc_body,
        grid=(_NBLK,),
        in_specs=[
            pl.BlockSpec((_BLK, _SW), lambda i: (i, 0)),
            pl.BlockSpec((DIM, DIM), lambda i: (0, 0)),
            pl.BlockSpec((1, DIM), lambda i: (0, 0)),
            pl.BlockSpec((DIM, DIM), lambda i: (0, 0)),
        ],
        out_specs=pl.BlockSpec((_BLK, DIM), lambda i: (i, 0)),
        out_shape=jax.ShapeDtypeStruct((2 * BATCH, DIM), jnp.float32),
        scratch_shapes=[
            pltpu.VMEM((DIM, DIM), jnp.float32),
            pltpu.VMEM((1, DIM), jnp.float32),
        ],
    )(staging, W_enc, b2, W_dec)


def kernel(src_id, tgt_id, emb_src, emb_tgt, W_enc, b_enc, W_dec):
    src_id = src_id.astype(jnp.int32)
    tgt_id = tgt_id.astype(jnp.int32)
    staging = _sc_gather(src_id, tgt_id, emb_src, emb_tgt)
    b2 = b_enc.reshape(1, DIM)
    return _tc_transform(staging, W_enc, b2, W_dec)


# speedup vs baseline: 2.5437x; 1.2389x over previous
import functools

import jax
import jax.numpy as jnp
from jax import lax
from jax.experimental import pallas as pl
from jax.experimental.pallas import tpu as pltpu
from jax.experimental.pallas import tpu_sc as plsc

VOCAB = 100000
DIM = 300
BATCH = 16384

_NC = 2
_NS = 16
_NW = _NC * _NS
_BPW = BATCH // _NW
_CH = 128
_NCH = _BPW // _CH
_KG = 16
_NG = _CH // _KG
_MAIN = 256
_TAIL = DIM - _MAIN
_SW = 384

_mesh = plsc.VectorSubcoreMesh(core_axis_name="c", subcore_axis_name="s")


@functools.partial(
    pl.kernel,
    mesh=_mesh,
    out_type=jax.ShapeDtypeStruct((BATCH, _SW), jnp.float32),
    scratch_types=[
        pltpu.VMEM((_CH,), jnp.int32),
        pltpu.VMEM((_CH, _MAIN), jnp.float32),
        pltpu.VMEM((_KG, 8, _TAIL), jnp.float32),
        pltpu.VMEM((_CH, 128), jnp.float32),
        pltpu.SemaphoreType.DMA,
        pltpu.SemaphoreType.DMA,
    ],
)
def _sc_gather(ids, tab, out_hbm, idx_v, main_v, blk_v, tail_v, sem_m, sem_t):
    wid = lax.axis_index("s") * _NC + lax.axis_index("c")
    base = wid * _BPW
    for j in range(_NCH):
        pltpu.sync_copy(ids.at[pl.ds(base + j * _CH, _CH)], idx_v)
        main_cp = pltpu.async_copy(
            tab.at[idx_v, pl.ds(0, _MAIN)], main_v, sem_m)

        def tail_group(g, _):
            gidx = idx_v[pl.ds(g * _KG, _KG)]
            for k in range(_KG):
                idx = gidx[k]
                i8 = pl.multiple_of((idx // 8) * 8, 8)
                pltpu.async_copy(
                    tab.at[pl.ds(i8, 8), pl.ds(_MAIN, _TAIL)],
                    blk_v.at[k], sem_t)
            for k in range(_KG):
                pltpu.make_async_copy(
                    tab.at[pl.ds(0, 8), pl.ds(_MAIN, _TAIL)],
                    blk_v.at[k], sem_t).wait()
            for k in range(_KG):
                idx = gidx[k]
                r = lax.rem(idx, 8)
                row = g * _KG + k
                tail_v[row, pl.ds(0, 16)] = blk_v[k, r, pl.ds(0, 16)]
                tail_v[row, pl.ds(16, 16)] = blk_v[k, r, pl.ds(16, 16)]
                tail_v[row, pl.ds(_TAIL - 16, 16)] = (
                    blk_v[k, r, pl.ds(_TAIL - 16, 16)])
            return 0

        lax.fori_loop(0, _NG, tail_group, 0)
        main_cp.wait()
        ob = base + j * _CH
        pltpu.sync_copy(main_v, out_hbm.at[pl.ds(ob, _CH), pl.ds(0, _MAIN)])
        pltpu.sync_copy(tail_v,
                        out_hbm.at[pl.ds(ob, _CH), pl.ds(_MAIN, 128)])


_BLK = 2048
_NTOP = BATCH // _BLK
_NBLK = 2 * _NTOP


def _tc_body(xs_ref, xt_ref, we_ref, b_ref, wd_ref, o_ref, wc_ref, bc_ref):
    i = pl.program_id(0)

    @pl.when(i == 0)
    def _():
        wc_ref[...] = lax.dot_general(
            we_ref[...], wd_ref[...], (((0,), (0,)), ((), ())),
            preferred_element_type=jnp.float32)
        bc_ref[...] = lax.dot_general(
            wd_ref[...], b_ref[...], (((0,), (1,)), ((), ())),
            preferred_element_type=jnp.float32)

    @pl.when(i < _NTOP)
    def _():
        o_ref[...] = lax.dot_general(
            wc_ref[...], xs_ref[:, :DIM], (((0,), (1,)), ((), ())),
            preferred_element_type=jnp.float32) + bc_ref[...]

    @pl.when(i >= _NTOP)
    def _():
        o_ref[...] = xt_ref[:, :DIM].T


def _tc_transform(stag_src, stag_tgt, W_enc, b2, W_dec):
    return pl.pallas_call(
        _tc_body,
        grid=(_NBLK,),
        in_specs=[
            pl.BlockSpec((_BLK, _SW), lambda i: (jnp.minimum(i, _NTOP - 1), 0)),
            pl.BlockSpec((_BLK, _SW),
                         lambda i: (jnp.maximum(i - _NTOP, 0), 0)),
            pl.BlockSpec((DIM, DIM), lambda i: (0, 0)),
            pl.BlockSpec((1, DIM), lambda i: (0, 0)),
            pl.BlockSpec((DIM, DIM), lambda i: (0, 0)),
        ],
        out_specs=pl.BlockSpec((DIM, _BLK), lambda i: (0, i)),
        out_shape=jax.ShapeDtypeStruct((DIM, 2 * BATCH), jnp.float32),
        scratch_shapes=[
            pltpu.VMEM((DIM, DIM), jnp.float32),
            pltpu.VMEM((DIM, 1), jnp.float32),
        ],
    )(stag_src, stag_tgt, W_enc, b2, W_dec)


def kernel(src_id, tgt_id, emb_src, emb_tgt, W_enc, b_enc, W_dec):
    src_id = src_id.astype(jnp.int32)
    tgt_id = tgt_id.astype(jnp.int32)
    stag_src = _sc_gather(src_id, emb_src)
    stag_tgt = _sc_gather(tgt_id, emb_tgt)
    b2 = b_enc.reshape(1, DIM)
    out_t = _tc_transform(stag_src, stag_tgt, W_enc, b2, W_dec)
    return out_t.T
